# baseline (speedup 1.0000x reference)
import jax
import jax.numpy as jnp
from jax.experimental import pallas as pl
from jax.experimental.pallas import tpu as pltpu

N = 10000
NP = 10240
C = 128
RB = 256
CW = 512
KC = 4
KM_ITERS = 10
BIGF = 3.0e38


def _fc1_body(x_ref, w_ref, b_ref, xf_ref, b2_ref):
    xf = jax.lax.dot_general(x_ref[...], w_ref[...], (((1,), (1,)), ((), ())),
                             preferred_element_type=jnp.float32) + b_ref[...]
    xf_ref[...] = xf
    b2_ref[...] = jnp.sum(xf * xf, axis=1, keepdims=True)


def _fc1(x, w, b):
    nrb = NP // RB
    return pl.pallas_call(
        _fc1_body,
        grid=(nrb,),
        in_specs=[pl.BlockSpec((RB, C), lambda r: (r, 0)),
                  pl.BlockSpec((C, C), lambda r: (0, 0)),
                  pl.BlockSpec((1, C), lambda r: (0, 0))],
        out_specs=[pl.BlockSpec((RB, C), lambda r: (r, 0)),
                   pl.BlockSpec((RB, 1), lambda r: (r, 0))],
        out_shape=[jax.ShapeDtypeStruct((NP, C), jnp.float32),
                   jax.ShapeDtypeStruct((NP, 1), jnp.float32)],
    )(x, w, b.reshape(1, C))


def _knn_body(rb_ref, a2_ref, xf_ref, b2r_ref, nbr_ref, rv_ref, ri_ref):
    c = pl.program_id(1)

    @pl.when(c == 0)
    def _():
        rv_ref[...] = jnp.full((RB, 8), BIGF, jnp.float32)
        ri_ref[...] = jnp.full((RB, 8), NP, jnp.int32)

    xfc = xf_ref[pl.ds(c * CW, CW), :]
    b2c = b2r_ref[:, pl.ds(c * CW, CW)]
    prod = jax.lax.dot_general(rb_ref[...], xfc, (((1,), (1,)), ((), ())),
                               preferred_element_type=jnp.float32)
    d = (a2_ref[...] + b2c) - 2.0 * prod
    colid = jax.lax.broadcasted_iota(jnp.int32, (RB, CW), 1) + c * CW
    cv = jnp.concatenate([rv_ref[...], d], axis=1)
    ci = jnp.concatenate([ri_ref[...], colid], axis=1)
    vs, idxs = [], []
    for t in range(3):
        m = jnp.min(cv, axis=1, keepdims=True)
        im = jnp.min(jnp.where(cv == m, ci, NP), axis=1, keepdims=True)
        vs.append(m)
        idxs.append(im)
        if t < 2:
            cv = jnp.where(ci == im, BIGF, cv)
    rv_ref[...] = jnp.concatenate(
        vs + [jnp.full((RB, 5), BIGF, jnp.float32)], axis=1)
    new_ri = jnp.concatenate(
        idxs + [jnp.full((RB, 5), NP, jnp.int32)], axis=1)
    ri_ref[...] = new_ri
    nbr_ref[...] = new_ri


def _knn(xf, b2c, b2r):
    nrb, ncw = NP // RB, NP // CW
    return pl.pallas_call(
        _knn_body,
        grid=(nrb, ncw),
        in_specs=[pl.BlockSpec((RB, C), lambda r, c: (r, 0)),
                  pl.BlockSpec((RB, 1), lambda r, c: (r, 0)),
                  pl.BlockSpec((NP, C), lambda r, c: (0, 0)),
                  pl.BlockSpec((1, NP), lambda r, c: (0, 0))],
        out_specs=pl.BlockSpec((RB, 8), lambda r, c: (r, 0)),
        out_shape=jax.ShapeDtypeStruct((NP, 8), jnp.int32),
        scratch_shapes=[pltpu.VMEM((RB, 8), jnp.float32),
                        pltpu.VMEM((RB, 8), jnp.int32)],
    )(xf, b2c, xf, b2r)


def _km_assign(xf, cent):
    c2 = jnp.sum(cent * cent, axis=1, keepdims=True)
    dots = jax.lax.dot_general(cent, xf, (((1,), (1,)), ((), ())),
                               preferred_element_type=jnp.float32)
    dc = c2 - 2.0 * dots
    rowi = jax.lax.broadcasted_iota(jnp.int32, (8, NP), 0)
    dc = jnp.where(rowi < KC, dc, BIGF)
    m = jnp.min(dc, axis=0, keepdims=True)
    sel = jnp.min(jnp.where(dc == m, rowi, 8), axis=0, keepdims=True)
    coli = jax.lax.broadcasted_iota(jnp.int32, (8, NP), 1)
    onehot = jnp.where((rowi == sel) & (coli < N), 1.0, 0.0)
    return onehot


def _km_step(onehot, xf):
    sums = jax.lax.dot_general(onehot, xf, (((1,), (0,)), ((), ())),
                               preferred_element_type=jnp.float32)
    cnts = jnp.sum(onehot, axis=1, keepdims=True)
    return sums / jnp.maximum(cnts, 1.0)


def _kmeans_body(xf_ref, kmc_ref):
    xf = xf_ref[...]
    cent0 = jnp.concatenate(
        [xf[0:KC, :], jnp.zeros((8 - KC, C), jnp.float32)], axis=0)

    def iter_fn(_, cent):
        return _km_step(_km_assign(xf, cent), xf)

    cent = jax.lax.fori_loop(0, KM_ITERS, iter_fn, cent0)
    onehot = _km_assign(xf, cent)
    x1_km = _km_step(onehot, xf)
    kmc_ref[...] = jax.lax.dot_general(
        onehot, x1_km, (((0,), (0,)), ((), ())),
        preferred_element_type=jnp.float32)


def _kmeans(xf):
    return pl.pallas_call(
        _kmeans_body,
        grid=(1,),
        in_specs=[pl.BlockSpec((NP, C), lambda i: (0, 0))],
        out_specs=pl.BlockSpec((NP, C), lambda i: (0, 0)),
        out_shape=jax.ShapeDtypeStruct((NP, C), jnp.float32),
    )(xf)


def _mp_body(nbr_ref, xf_ref, agg_ref, deg_ref):
    r = pl.program_id(0)

    @pl.when(r == 0)
    def _():
        agg_ref[...] = jnp.zeros_like(agg_ref)
        deg_ref[...] = jnp.zeros_like(deg_ref)

    nb = nbr_ref[...]
    rowi = jax.lax.broadcasted_iota(jnp.int32, (NP, RB), 0)
    colg = jax.lax.broadcasted_iota(jnp.int32, (NP, RB), 1) + r * RB
    g = jnp.zeros((NP, RB), jnp.float32)
    for j in range(3):
        g += jnp.where((rowi == nb[:, j][None, :]) & (colg < N), 1.0, 0.0)
    p = jax.lax.dot_general(g, xf_ref[...], (((0,), (0,)), ((), ())),
                            preferred_element_type=jnp.float32)
    agg_ref[...] += jax.lax.dot_general(
        g, p * (1.0 / 3.0), (((1,), (0,)), ((), ())),
        preferred_element_type=jnp.float32)
    deg_ref[...] += jnp.sum(g, axis=1, keepdims=True)


def _mp(nbr8, xf):
    nrb = NP // RB
    return pl.pallas_call(
        _mp_body,
        grid=(nrb,),
        in_specs=[pl.BlockSpec((RB, 8), lambda r: (r, 0)),
                  pl.BlockSpec((NP, C), lambda r: (0, 0))],
        out_specs=[pl.BlockSpec((NP, C), lambda r: (0, 0)),
                   pl.BlockSpec((NP, 1), lambda r: (0, 0))],
        out_shape=[jax.ShapeDtypeStruct((NP, C), jnp.float32),
                   jax.ShapeDtypeStruct((NP, 1), jnp.float32)],
    )(nbr8, xf)


def _fc2_body(agg_ref, deg_ref, kmc_ref, w_ref, b_ref, out_ref):
    x0 = (agg_ref[...] + kmc_ref[...]) / (deg_ref[...] + 1.0)
    out_ref[...] = jax.lax.dot_general(
        x0, w_ref[...], (((1,), (1,)), ((), ())),
        preferred_element_type=jnp.float32) + b_ref[...]


def _fc2(agg, deg, kmc, w, b):
    nrb = NP // RB
    return pl.pallas_call(
        _fc2_body,
        grid=(nrb,),
        in_specs=[pl.BlockSpec((RB, C), lambda r: (r, 0)),
                  pl.BlockSpec((RB, 1), lambda r: (r, 0)),
                  pl.BlockSpec((RB, C), lambda r: (r, 0)),
                  pl.BlockSpec((C, C), lambda r: (0, 0)),
                  pl.BlockSpec((1, C), lambda r: (0, 0))],
        out_specs=pl.BlockSpec((RB, C), lambda r: (r, 0)),
        out_shape=jax.ShapeDtypeStruct((NP, C), jnp.float32),
    )(agg, deg, kmc, w, b.reshape(1, C))


def _pool_body(h_ref, w_ref, b_ref, out_ref):
    rowi = jax.lax.broadcasted_iota(jnp.int32, (NP, C), 0)
    hm = jnp.where(rowi < N, h_ref[...], -BIGF)
    pooled = jnp.max(hm, axis=0, keepdims=True)
    out_ref[...] = jax.lax.dot_general(
        pooled, w_ref[...], (((1,), (1,)), ((), ())),
        preferred_element_type=jnp.float32) + b_ref[...]


def _pool(h, w, b):
    return pl.pallas_call(
        _pool_body,
        grid=(1,),
        in_specs=[pl.BlockSpec((NP, C), lambda i: (0, 0)),
                  pl.BlockSpec((C, C), lambda i: (0, 0)),
                  pl.BlockSpec((1, C), lambda i: (0, 0))],
        out_specs=pl.BlockSpec((1, C), lambda i: (0, 0)),
        out_shape=jax.ShapeDtypeStruct((1, C), jnp.float32),
    )(h, w, b.reshape(1, C))


def _layer(xp, w1, b1, w2, b2):
    xf, b2col = _fc1(xp, w1, b1)
    col = jnp.arange(NP, dtype=jnp.int32)
    b2row = jnp.where(col[None, :] < N, b2col.reshape(1, NP), BIGF)
    nbr8 = _knn(xf, b2col, b2row)
    agg, deg = _mp(nbr8, xf)
    kmc = _kmeans(xf)
    return _fc2(agg, deg, kmc, w2, b2)


def kernel(x_0, fc1_w0, fc1_b0, fc2_w0, fc2_b0, fc1_w1, fc1_b1,
           fc2_w1, fc2_b1, lin_w, lin_b):
    xp = jnp.pad(x_0, ((0, NP - N), (0, 0)))
    h = _layer(xp, fc1_w0, fc1_b0, fc2_w0, fc2_b0)
    h = _layer(h, fc1_w1, fc1_b1, fc2_w1, fc2_b1)
    return _pool(h, lin_w, lin_b).reshape(C)


# speedup vs baseline: 1.3658x; 1.3658x over previous
import jax
import jax.numpy as jnp
from jax.experimental import pallas as pl
from jax.experimental.pallas import tpu as pltpu

N = 10000
NP = 10240
C = 128
RB = 256
CW = 1024
KC = 4
KM_ITERS = 10
BIGF = 3.0e38


def _fc1_body(x_ref, w_ref, b_ref, xf_ref, b2_ref):
    xf = jax.lax.dot_general(x_ref[...], w_ref[...], (((1,), (1,)), ((), ())),
                             preferred_element_type=jnp.float32) + b_ref[...]
    xf_ref[...] = xf
    b2_ref[...] = jnp.sum(xf * xf, axis=1, keepdims=True)


def _fc1(x, w, b):
    nrb = NP // RB
    return pl.pallas_call(
        _fc1_body,
        grid=(nrb,),
        in_specs=[pl.BlockSpec((RB, C), lambda r: (r, 0)),
                  pl.BlockSpec((C, C), lambda r: (0, 0)),
                  pl.BlockSpec((1, C), lambda r: (0, 0))],
        out_specs=[pl.BlockSpec((RB, C), lambda r: (r, 0)),
                   pl.BlockSpec((RB, 1), lambda r: (r, 0))],
        out_shape=[jax.ShapeDtypeStruct((NP, C), jnp.float32),
                   jax.ShapeDtypeStruct((NP, 1), jnp.float32)],
    )(x, w, b.reshape(1, C))


def _knn_body(rb_ref, a2_ref, xfc_ref, b2c_ref, nbr_ref, rv_ref, ri_ref):
    c = pl.program_id(1)

    @pl.when(c == 0)
    def _():
        rv_ref[...] = jnp.full((RB, 8), BIGF, jnp.float32)
        ri_ref[...] = jnp.full((RB, 8), NP, jnp.int32)

    prod = jax.lax.dot_general(rb_ref[...], xfc_ref[...],
                               (((1,), (1,)), ((), ())),
                               preferred_element_type=jnp.float32)
    d = (a2_ref[...] + b2c_ref[...]) - 2.0 * prod
    colid = jax.lax.broadcasted_iota(jnp.int32, (RB, CW), 1) + c * CW
    cv = jnp.concatenate([rv_ref[...], d], axis=1)
    ci = jnp.concatenate([ri_ref[...], colid], axis=1)
    vs, idxs = [], []
    for t in range(3):
        m = jnp.min(cv, axis=1, keepdims=True)
        im = jnp.min(jnp.where(cv == m, ci, NP), axis=1, keepdims=True)
        vs.append(m)
        idxs.append(im)
        if t < 2:
            cv = jnp.where(ci == im, BIGF, cv)
    rv_ref[...] = jnp.concatenate(
        vs + [jnp.full((RB, 5), BIGF, jnp.float32)], axis=1)
    new_ri = jnp.concatenate(
        idxs + [jnp.full((RB, 5), NP, jnp.int32)], axis=1)
    ri_ref[...] = new_ri
    nbr_ref[...] = new_ri


def _knn(xf, b2c, b2r):
    nrb, ncw = NP // RB, NP // CW
    return pl.pallas_call(
        _knn_body,
        grid=(nrb, ncw),
        in_specs=[pl.BlockSpec((RB, C), lambda r, c: (r, 0)),
                  pl.BlockSpec((RB, 1), lambda r, c: (r, 0)),
                  pl.BlockSpec((CW, C), lambda r, c: (c, 0)),
                  pl.BlockSpec((1, CW), lambda r, c: (0, c))],
        out_specs=pl.BlockSpec((RB, 8), lambda r, c: (r, 0)),
        out_shape=jax.ShapeDtypeStruct((NP, 8), jnp.int32),
        scratch_shapes=[pltpu.VMEM((RB, 8), jnp.float32),
                        pltpu.VMEM((RB, 8), jnp.int32)],
    )(xf, b2c, xf, b2r)


def _km_assign(xf, cent):
    c2 = jnp.sum(cent * cent, axis=1, keepdims=True)
    dots = jax.lax.dot_general(cent, xf, (((1,), (1,)), ((), ())),
                               preferred_element_type=jnp.float32)
    dc = c2 - 2.0 * dots
    rowi = jax.lax.broadcasted_iota(jnp.int32, (8, NP), 0)
    dc = jnp.where(rowi < KC, dc, BIGF)
    m = jnp.min(dc, axis=0, keepdims=True)
    sel = jnp.min(jnp.where(dc == m, rowi, 8), axis=0, keepdims=True)
    coli = jax.lax.broadcasted_iota(jnp.int32, (8, NP), 1)
    onehot = jnp.where((rowi == sel) & (coli < N), 1.0, 0.0)
    return onehot


def _km_step(onehot, xf):
    sums = jax.lax.dot_general(onehot, xf, (((1,), (0,)), ((), ())),
                               preferred_element_type=jnp.float32)
    cnts = jnp.sum(onehot, axis=1, keepdims=True)
    return sums / jnp.maximum(cnts, 1.0)


def _kmeans_body(xf_ref, kmc_ref):
    xf = xf_ref[...]
    cent0 = jnp.concatenate(
        [xf[0:KC, :], jnp.zeros((8 - KC, C), jnp.float32)], axis=0)

    def iter_fn(_, cent):
        return _km_step(_km_assign(xf, cent), xf)

    cent = jax.lax.fori_loop(0, KM_ITERS, iter_fn, cent0)
    onehot = _km_assign(xf, cent)
    x1_km = _km_step(onehot, xf)
    kmc_ref[...] = jax.lax.dot_general(
        onehot, x1_km, (((0,), (0,)), ((), ())),
        preferred_element_type=jnp.float32)


def _kmeans(xf):
    return pl.pallas_call(
        _kmeans_body,
        grid=(1,),
        in_specs=[pl.BlockSpec((NP, C), lambda i: (0, 0))],
        out_specs=pl.BlockSpec((NP, C), lambda i: (0, 0)),
        out_shape=jax.ShapeDtypeStruct((NP, C), jnp.float32),
    )(xf)


def _mp_body(nbr_ref, xf_ref, agg_ref, deg_ref):
    r = pl.program_id(0)

    @pl.when(r == 0)
    def _():
        agg_ref[...] = jnp.zeros_like(agg_ref)
        deg_ref[...] = jnp.zeros_like(deg_ref)

    nb = nbr_ref[...]
    loc = jax.lax.broadcasted_iota(jnp.int32, (RB, 8), 0) + r * RB
    nb = jnp.where(loc < N, nb, NP)
    rowi = jax.lax.broadcasted_iota(jnp.int32, (NP, RB), 0)
    m = (rowi == nb[:, 0][None, :]) | (rowi == nb[:, 1][None, :]) \
        | (rowi == nb[:, 2][None, :])
    g = jnp.where(m, 1.0, 0.0)
    p = jax.lax.dot_general(g, xf_ref[...], (((0,), (0,)), ((), ())),
                            preferred_element_type=jnp.float32)
    agg_ref[...] += jax.lax.dot_general(
        g, p * (1.0 / 3.0), (((1,), (0,)), ((), ())),
        preferred_element_type=jnp.float32)
    deg_ref[...] += jnp.sum(g, axis=1, keepdims=True)


def _mp(nbr8, xf):
    nrb = NP // RB
    return pl.pallas_call(
        _mp_body,
        grid=(nrb,),
        in_specs=[pl.BlockSpec((RB, 8), lambda r: (r, 0)),
                  pl.BlockSpec((NP, C), lambda r: (0, 0))],
        out_specs=[pl.BlockSpec((NP, C), lambda r: (0, 0)),
                   pl.BlockSpec((NP, 1), lambda r: (0, 0))],
        out_shape=[jax.ShapeDtypeStruct((NP, C), jnp.float32),
                   jax.ShapeDtypeStruct((NP, 1), jnp.float32)],
    )(nbr8, xf)


def _fc2_body(agg_ref, deg_ref, kmc_ref, w_ref, b_ref, out_ref):
    x0 = (agg_ref[...] + kmc_ref[...]) / (deg_ref[...] + 1.0)
    out_ref[...] = jax.lax.dot_general(
        x0, w_ref[...], (((1,), (1,)), ((), ())),
        preferred_element_type=jnp.float32) + b_ref[...]


def _fc2(agg, deg, kmc, w, b):
    nrb = NP // RB
    return pl.pallas_call(
        _fc2_body,
        grid=(nrb,),
        in_specs=[pl.BlockSpec((RB, C), lambda r: (r, 0)),
                  pl.BlockSpec((RB, 1), lambda r: (r, 0)),
                  pl.BlockSpec((RB, C), lambda r: (r, 0)),
                  pl.BlockSpec((C, C), lambda r: (0, 0)),
                  pl.BlockSpec((1, C), lambda r: (0, 0))],
        out_specs=pl.BlockSpec((RB, C), lambda r: (r, 0)),
        out_shape=jax.ShapeDtypeStruct((NP, C), jnp.float32),
    )(agg, deg, kmc, w, b.reshape(1, C))


def _pool_body(h_ref, w_ref, b_ref, out_ref):
    rowi = jax.lax.broadcasted_iota(jnp.int32, (NP, C), 0)
    hm = jnp.where(rowi < N, h_ref[...], -BIGF)
    pooled = jnp.max(hm, axis=0, keepdims=True)
    out_ref[...] = jax.lax.dot_general(
        pooled, w_ref[...], (((1,), (1,)), ((), ())),
        preferred_element_type=jnp.float32) + b_ref[...]


def _pool(h, w, b):
    return pl.pallas_call(
        _pool_body,
        grid=(1,),
        in_specs=[pl.BlockSpec((NP, C), lambda i: (0, 0)),
                  pl.BlockSpec((C, C), lambda i: (0, 0)),
                  pl.BlockSpec((1, C), lambda i: (0, 0))],
        out_specs=pl.BlockSpec((1, C), lambda i: (0, 0)),
        out_shape=jax.ShapeDtypeStruct((1, C), jnp.float32),
    )(h, w, b.reshape(1, C))


def _layer(xp, w1, b1, w2, b2):
    xf, b2col = _fc1(xp, w1, b1)
    col = jnp.arange(NP, dtype=jnp.int32)
    b2row = jnp.where(col[None, :] < N, b2col.reshape(1, NP), BIGF)
    nbr8 = _knn(xf, b2col, b2row)
    agg, deg = _mp(nbr8, xf)
    kmc = _kmeans(xf)
    return _fc2(agg, deg, kmc, w2, b2)


def kernel(x_0, fc1_w0, fc1_b0, fc2_w0, fc2_b0, fc1_w1, fc1_b1,
           fc2_w1, fc2_b1, lin_w, lin_b):
    xp = jnp.pad(x_0, ((0, NP - N), (0, 0)))
    h = _layer(xp, fc1_w0, fc1_b0, fc2_w0, fc2_b0)
    h = _layer(h, fc1_w1, fc1_b1, fc2_w1, fc2_b1)
    return _pool(h, lin_w, lin_b).reshape(C)


# speedup vs baseline: 1.5815x; 1.1580x over previous
import jax
import jax.numpy as jnp
from jax.experimental import pallas as pl
from jax.experimental.pallas import tpu as pltpu

N = 10000
NP = 10240
C = 128
RB = 256
CW = 1024
RBK = 512
RBM = 256
KC = 4
KM_ITERS = 10
BIGF = 3.0e38


def _fc1_body(x_ref, w_ref, b_ref, xf_ref, b2_ref):
    xf = jax.lax.dot_general(x_ref[...], w_ref[...], (((1,), (1,)), ((), ())),
                             preferred_element_type=jnp.float32) + b_ref[...]
    xf_ref[...] = xf
    b2_ref[...] = jnp.sum(xf * xf, axis=1, keepdims=True)


def _fc1(x, w, b):
    nrb = NP // RB
    return pl.pallas_call(
        _fc1_body,
        grid=(nrb,),
        in_specs=[pl.BlockSpec((RB, C), lambda r: (r, 0)),
                  pl.BlockSpec((C, C), lambda r: (0, 0)),
                  pl.BlockSpec((1, C), lambda r: (0, 0))],
        out_specs=[pl.BlockSpec((RB, C), lambda r: (r, 0)),
                   pl.BlockSpec((RB, 1), lambda r: (r, 0))],
        out_shape=[jax.ShapeDtypeStruct((NP, C), jnp.float32),
                   jax.ShapeDtypeStruct((NP, 1), jnp.float32)],
    )(x, w, b.reshape(1, C))


def _knn_body(rb_ref, a2_ref, xfc_ref, b2c_ref, nbr_ref, rv_ref, ri_ref):
    c = pl.program_id(1)

    @pl.when(c == 0)
    def _():
        rv_ref[...] = jnp.full((RBK, 8), BIGF, jnp.float32)
        ri_ref[...] = jnp.full((RBK, 8), NP, jnp.int32)

    prod = jax.lax.dot_general(rb_ref[...], xfc_ref[...],
                               (((1,), (1,)), ((), ())),
                               preferred_element_type=jnp.float32)
    d = (a2_ref[...] + b2c_ref[...]) - 2.0 * prod
    colid = jax.lax.broadcasted_iota(jnp.int32, (RBK, CW), 1) + c * CW
    cv = jnp.concatenate([rv_ref[...], d], axis=1)
    ci = jnp.concatenate([ri_ref[...], colid], axis=1)
    vs, idxs = [], []
    for t in range(3):
        m = jnp.min(cv, axis=1, keepdims=True)
        im = jnp.min(jnp.where(cv == m, ci, NP), axis=1, keepdims=True)
        vs.append(m)
        idxs.append(im)
        if t < 2:
            cv = jnp.where(ci == im, BIGF, cv)
    rv_ref[...] = jnp.concatenate(
        vs + [jnp.full((RBK, 5), BIGF, jnp.float32)], axis=1)
    new_ri = jnp.concatenate(
        idxs + [jnp.full((RBK, 5), NP, jnp.int32)], axis=1)
    ri_ref[...] = new_ri
    nbr_ref[...] = new_ri


def _knn(xf, b2c, b2r):
    nrb, ncw = NP // RBK, NP // CW
    return pl.pallas_call(
        _knn_body,
        grid=(nrb, ncw),
        in_specs=[pl.BlockSpec((RBK, C), lambda r, c: (r, 0)),
                  pl.BlockSpec((RBK, 1), lambda r, c: (r, 0)),
                  pl.BlockSpec((CW, C), lambda r, c: (c, 0)),
                  pl.BlockSpec((1, CW), lambda r, c: (0, c))],
        out_specs=pl.BlockSpec((RBK, 8), lambda r, c: (r, 0)),
        out_shape=jax.ShapeDtypeStruct((NP, 8), jnp.int32),
        scratch_shapes=[pltpu.VMEM((RBK, 8), jnp.float32),
                        pltpu.VMEM((RBK, 8), jnp.int32)],
    )(xf, b2c, xf, b2r)


def _km_assign(xf, cent):
    c2 = jnp.sum(cent * cent, axis=1, keepdims=True)
    dots = jax.lax.dot_general(cent, xf, (((1,), (1,)), ((), ())),
                               preferred_element_type=jnp.float32)
    dc = c2 - 2.0 * dots
    rowi = jax.lax.broadcasted_iota(jnp.int32, (8, NP), 0)
    dc = jnp.where(rowi < KC, dc, BIGF)
    m = jnp.min(dc, axis=0, keepdims=True)
    sel = jnp.min(jnp.where(dc == m, rowi, 8), axis=0, keepdims=True)
    coli = jax.lax.broadcasted_iota(jnp.int32, (8, NP), 1)
    onehot = jnp.where((rowi == sel) & (coli < N), 1.0, 0.0)
    return onehot


def _km_step(onehot, xf):
    sums = jax.lax.dot_general(onehot, xf, (((1,), (0,)), ((), ())),
                               preferred_element_type=jnp.float32)
    cnts = jnp.sum(onehot, axis=1, keepdims=True)
    return sums / jnp.maximum(cnts, 1.0)


def _kmeans_body(xf_ref, kmc_ref):
    xf = xf_ref[...]
    cent0 = jnp.concatenate(
        [xf[0:KC, :], jnp.zeros((8 - KC, C), jnp.float32)], axis=0)

    def iter_fn(_, cent):
        return _km_step(_km_assign(xf, cent), xf)

    cent = jax.lax.fori_loop(0, KM_ITERS, iter_fn, cent0)
    onehot = _km_assign(xf, cent)
    x1_km = _km_step(onehot, xf)
    kmc_ref[...] = jax.lax.dot_general(
        onehot, x1_km, (((0,), (0,)), ((), ())),
        preferred_element_type=jnp.float32)


def _kmeans(xf):
    return pl.pallas_call(
        _kmeans_body,
        grid=(1,),
        in_specs=[pl.BlockSpec((NP, C), lambda i: (0, 0))],
        out_specs=pl.BlockSpec((NP, C), lambda i: (0, 0)),
        out_shape=jax.ShapeDtypeStruct((NP, C), jnp.float32),
    )(xf)


def _mp_body(nbr_ref, xf_ref, agg_ref, deg_ref):
    r = pl.program_id(0)

    @pl.when(r == 0)
    def _():
        agg_ref[...] = jnp.zeros_like(agg_ref)
        deg_ref[...] = jnp.zeros_like(deg_ref)

    nb = nbr_ref[...]
    loc = jax.lax.broadcasted_iota(jnp.int32, (RBM, 8), 0) + r * RBM
    nb = jnp.where(loc < N, nb, NP)
    rowi = jax.lax.broadcasted_iota(jnp.int32, (NP, RBM), 0)
    m = (rowi == nb[:, 0][None, :]) | (rowi == nb[:, 1][None, :]) \
        | (rowi == nb[:, 2][None, :])
    g = jnp.where(m, 1.0, 0.0)
    p = jax.lax.dot_general(g, xf_ref[...], (((0,), (0,)), ((), ())),
                            preferred_element_type=jnp.float32)
    agg_ref[...] += jax.lax.dot_general(
        g, p * (1.0 / 3.0), (((1,), (0,)), ((), ())),
        preferred_element_type=jnp.float32)
    deg_ref[...] += jnp.sum(g, axis=1, keepdims=True)


def _mp(nbr8, xf):
    nrb = NP // RBM
    return pl.pallas_call(
        _mp_body,
        grid=(nrb,),
        in_specs=[pl.BlockSpec((RBM, 8), lambda r: (r, 0)),
                  pl.BlockSpec((NP, C), lambda r: (0, 0))],
        out_specs=[pl.BlockSpec((NP, C), lambda r: (0, 0)),
                   pl.BlockSpec((NP, 1), lambda r: (0, 0))],
        out_shape=[jax.ShapeDtypeStruct((NP, C), jnp.float32),
                   jax.ShapeDtypeStruct((NP, 1), jnp.float32)],
    )(nbr8, xf)


def _fc2_body(agg_ref, deg_ref, kmc_ref, w_ref, b_ref, out_ref):
    x0 = (agg_ref[...] + kmc_ref[...]) / (deg_ref[...] + 1.0)
    out_ref[...] = jax.lax.dot_general(
        x0, w_ref[...], (((1,), (1,)), ((), ())),
        preferred_element_type=jnp.float32) + b_ref[...]


def _fc2(agg, deg, kmc, w, b):
    nrb = NP // RB
    return pl.pallas_call(
        _fc2_body,
        grid=(nrb,),
        in_specs=[pl.BlockSpec((RB, C), lambda r: (r, 0)),
                  pl.BlockSpec((RB, 1), lambda r: (r, 0)),
                  pl.BlockSpec((RB, C), lambda r: (r, 0)),
                  pl.BlockSpec((C, C), lambda r: (0, 0)),
                  pl.BlockSpec((1, C), lambda r: (0, 0))],
        out_specs=pl.BlockSpec((RB, C), lambda r: (r, 0)),
        out_shape=jax.ShapeDtypeStruct((NP, C), jnp.float32),
    )(agg, deg, kmc, w, b.reshape(1, C))


def _pool_body(h_ref, w_ref, b_ref, out_ref):
    rowi = jax.lax.broadcasted_iota(jnp.int32, (NP, C), 0)
    hm = jnp.where(rowi < N, h_ref[...], -BIGF)
    pooled = jnp.max(hm, axis=0, keepdims=True)
    out_ref[...] = jax.lax.dot_general(
        pooled, w_ref[...], (((1,), (1,)), ((), ())),
        preferred_element_type=jnp.float32) + b_ref[...]


def _pool(h, w, b):
    return pl.pallas_call(
        _pool_body,
        grid=(1,),
        in_specs=[pl.BlockSpec((NP, C), lambda i: (0, 0)),
                  pl.BlockSpec((C, C), lambda i: (0, 0)),
                  pl.BlockSpec((1, C), lambda i: (0, 0))],
        out_specs=pl.BlockSpec((1, C), lambda i: (0, 0)),
        out_shape=jax.ShapeDtypeStruct((1, C), jnp.float32),
    )(h, w, b.reshape(1, C))


def _layer(xp, w1, b1, w2, b2):
    xf, b2col = _fc1(xp, w1, b1)
    col = jnp.arange(NP, dtype=jnp.int32)
    b2row = jnp.where(col[None, :] < N, b2col.reshape(1, NP), BIGF)
    nbr8 = _knn(xf, b2col, b2row)
    agg, deg = _mp(nbr8, xf)
    kmc = _kmeans(xf)
    return _fc2(agg, deg, kmc, w2, b2)


def kernel(x_0, fc1_w0, fc1_b0, fc2_w0, fc2_b0, fc1_w1, fc1_b1,
           fc2_w1, fc2_b1, lin_w, lin_b):
    xp = jnp.pad(x_0, ((0, NP - N), (0, 0)))
    h = _layer(xp, fc1_w0, fc1_b0, fc2_w0, fc2_b0)
    h = _layer(h, fc1_w1, fc1_b1, fc2_w1, fc2_b1)
    return _pool(h, lin_w, lin_b).reshape(C)


# speedup vs baseline: 1.6311x; 1.0314x over previous
import jax
import jax.numpy as jnp
from jax.experimental import pallas as pl
from jax.experimental.pallas import tpu as pltpu

N = 10000
NP = 10240
C = 128
RB = 256
CW = 1024
RBK = 1024
RBM = 256
KC = 4
KM_ITERS = 10
BIGF = 3.0e38


def _fc1_body(x_ref, w_ref, b_ref, xf_ref, b2_ref):
    xf = jax.lax.dot_general(x_ref[...], w_ref[...], (((1,), (1,)), ((), ())),
                             preferred_element_type=jnp.float32) + b_ref[...]
    xf_ref[...] = xf
    b2_ref[...] = jnp.sum(xf * xf, axis=1, keepdims=True)


def _fc1(x, w, b):
    nrb = NP // RB
    return pl.pallas_call(
        _fc1_body,
        grid=(nrb,),
        in_specs=[pl.BlockSpec((RB, C), lambda r: (r, 0)),
                  pl.BlockSpec((C, C), lambda r: (0, 0)),
                  pl.BlockSpec((1, C), lambda r: (0, 0))],
        out_specs=[pl.BlockSpec((RB, C), lambda r: (r, 0)),
                   pl.BlockSpec((RB, 1), lambda r: (r, 0))],
        out_shape=[jax.ShapeDtypeStruct((NP, C), jnp.float32),
                   jax.ShapeDtypeStruct((NP, 1), jnp.float32)],
    )(x, w, b.reshape(1, C))


def _knn_body(rb_ref, a2_ref, xfc_ref, b2c_ref, nbr_ref, rv_ref, ri_ref):
    c = pl.program_id(1)

    @pl.when(c == 0)
    def _():
        rv_ref[...] = jnp.full((RBK, 8), BIGF, jnp.float32)
        ri_ref[...] = jnp.full((RBK, 8), NP, jnp.int32)

    prod = jax.lax.dot_general(rb_ref[...], xfc_ref[...],
                               (((1,), (1,)), ((), ())),
                               preferred_element_type=jnp.float32)
    d = (a2_ref[...] + b2c_ref[...]) - 2.0 * prod
    colid = jax.lax.broadcasted_iota(jnp.int32, (RBK, CW), 1) + c * CW
    cv = jnp.concatenate([rv_ref[...], d], axis=1)
    ci = jnp.concatenate([ri_ref[...], colid], axis=1)
    vs, idxs = [], []
    for t in range(3):
        m = jnp.min(cv, axis=1, keepdims=True)
        im = jnp.min(jnp.where(cv == m, ci, NP), axis=1, keepdims=True)
        vs.append(m)
        idxs.append(im)
        if t < 2:
            cv = jnp.where(ci == im, BIGF, cv)
    rv_ref[...] = jnp.concatenate(
        vs + [jnp.full((RBK, 5), BIGF, jnp.float32)], axis=1)
    new_ri = jnp.concatenate(
        idxs + [jnp.full((RBK, 5), NP, jnp.int32)], axis=1)
    ri_ref[...] = new_ri
    nbr_ref[...] = new_ri


def _knn(xf, b2c, b2r):
    nrb, ncw = NP // RBK, NP // CW
    return pl.pallas_call(
        _knn_body,
        grid=(nrb, ncw),
        in_specs=[pl.BlockSpec((RBK, C), lambda r, c: (r, 0)),
                  pl.BlockSpec((RBK, 1), lambda r, c: (r, 0)),
                  pl.BlockSpec((CW, C), lambda r, c: (c, 0)),
                  pl.BlockSpec((1, CW), lambda r, c: (0, c))],
        out_specs=pl.BlockSpec((RBK, 8), lambda r, c: (r, 0)),
        out_shape=jax.ShapeDtypeStruct((NP, 8), jnp.int32),
        scratch_shapes=[pltpu.VMEM((RBK, 8), jnp.float32),
                        pltpu.VMEM((RBK, 8), jnp.int32)],
    )(xf, b2c, xf, b2r)


def _km_assign(xf, cent):
    c2 = jnp.sum(cent * cent, axis=1, keepdims=True)
    dots = jax.lax.dot_general(cent, xf, (((1,), (1,)), ((), ())),
                               preferred_element_type=jnp.float32)
    dc = c2 - 2.0 * dots
    rowi = jax.lax.broadcasted_iota(jnp.int32, (8, NP), 0)
    dc = jnp.where(rowi < KC, dc, BIGF)
    m = jnp.min(dc, axis=0, keepdims=True)
    sel = jnp.min(jnp.where(dc == m, rowi, 8), axis=0, keepdims=True)
    coli = jax.lax.broadcasted_iota(jnp.int32, (8, NP), 1)
    onehot = jnp.where((rowi == sel) & (coli < N), 1.0, 0.0)
    return onehot


def _km_step(onehot, xf):
    sums = jax.lax.dot_general(onehot, xf, (((1,), (0,)), ((), ())),
                               preferred_element_type=jnp.float32)
    cnts = jnp.sum(onehot, axis=1, keepdims=True)
    return sums / jnp.maximum(cnts, 1.0)


def _kmeans_body(xf_ref, kmc_ref):
    xf = xf_ref[...]
    cent0 = jnp.concatenate(
        [xf[0:KC, :], jnp.zeros((8 - KC, C), jnp.float32)], axis=0)

    def iter_fn(_, cent):
        return _km_step(_km_assign(xf, cent), xf)

    cent = jax.lax.fori_loop(0, KM_ITERS, iter_fn, cent0)
    onehot = _km_assign(xf, cent)
    x1_km = _km_step(onehot, xf)
    kmc_ref[...] = jax.lax.dot_general(
        onehot, x1_km, (((0,), (0,)), ((), ())),
        preferred_element_type=jnp.float32)


def _kmeans(xf):
    return pl.pallas_call(
        _kmeans_body,
        grid=(1,),
        in_specs=[pl.BlockSpec((NP, C), lambda i: (0, 0))],
        out_specs=pl.BlockSpec((NP, C), lambda i: (0, 0)),
        out_shape=jax.ShapeDtypeStruct((NP, C), jnp.float32),
    )(xf)


def _mp_body(nbr_ref, xf_ref, agg_ref, deg_ref):
    r = pl.program_id(0)

    @pl.when(r == 0)
    def _():
        agg_ref[...] = jnp.zeros_like(agg_ref)
        deg_ref[...] = jnp.zeros_like(deg_ref)

    nb = nbr_ref[...]
    loc = jax.lax.broadcasted_iota(jnp.int32, (RBM, 8), 0) + r * RBM
    nb = jnp.where(loc < N, nb, NP)
    rowi = jax.lax.broadcasted_iota(jnp.int32, (NP, RBM), 0)
    m = (rowi == nb[:, 0][None, :]) | (rowi == nb[:, 1][None, :]) \
        | (rowi == nb[:, 2][None, :])
    g = jnp.where(m, 1.0, 0.0)
    p = jax.lax.dot_general(g, xf_ref[...], (((0,), (0,)), ((), ())),
                            preferred_element_type=jnp.float32)
    agg_ref[...] += jax.lax.dot_general(
        g, p * (1.0 / 3.0), (((1,), (0,)), ((), ())),
        preferred_element_type=jnp.float32)
    deg_ref[...] += jnp.sum(g, axis=1, keepdims=True)


def _mp(nbr8, xf):
    nrb = NP // RBM
    return pl.pallas_call(
        _mp_body,
        grid=(nrb,),
        in_specs=[pl.BlockSpec((RBM, 8), lambda r: (r, 0)),
                  pl.BlockSpec((NP, C), lambda r: (0, 0))],
        out_specs=[pl.BlockSpec((NP, C), lambda r: (0, 0)),
                   pl.BlockSpec((NP, 1), lambda r: (0, 0))],
        out_shape=[jax.ShapeDtypeStruct((NP, C), jnp.float32),
                   jax.ShapeDtypeStruct((NP, 1), jnp.float32)],
    )(nbr8, xf)


def _fc2_body(agg_ref, deg_ref, kmc_ref, w_ref, b_ref, out_ref):
    x0 = (agg_ref[...] + kmc_ref[...]) / (deg_ref[...] + 1.0)
    out_ref[...] = jax.lax.dot_general(
        x0, w_ref[...], (((1,), (1,)), ((), ())),
        preferred_element_type=jnp.float32) + b_ref[...]


def _fc2(agg, deg, kmc, w, b):
    nrb = NP // RB
    return pl.pallas_call(
        _fc2_body,
        grid=(nrb,),
        in_specs=[pl.BlockSpec((RB, C), lambda r: (r, 0)),
                  pl.BlockSpec((RB, 1), lambda r: (r, 0)),
                  pl.BlockSpec((RB, C), lambda r: (r, 0)),
                  pl.BlockSpec((C, C), lambda r: (0, 0)),
                  pl.BlockSpec((1, C), lambda r: (0, 0))],
        out_specs=pl.BlockSpec((RB, C), lambda r: (r, 0)),
        out_shape=jax.ShapeDtypeStruct((NP, C), jnp.float32),
    )(agg, deg, kmc, w, b.reshape(1, C))


def _pool_body(h_ref, w_ref, b_ref, out_ref):
    rowi = jax.lax.broadcasted_iota(jnp.int32, (NP, C), 0)
    hm = jnp.where(rowi < N, h_ref[...], -BIGF)
    pooled = jnp.max(hm, axis=0, keepdims=True)
    out_ref[...] = jax.lax.dot_general(
        pooled, w_ref[...], (((1,), (1,)), ((), ())),
        preferred_element_type=jnp.float32) + b_ref[...]


def _pool(h, w, b):
    return pl.pallas_call(
        _pool_body,
        grid=(1,),
        in_specs=[pl.BlockSpec((NP, C), lambda i: (0, 0)),
                  pl.BlockSpec((C, C), lambda i: (0, 0)),
                  pl.BlockSpec((1, C), lambda i: (0, 0))],
        out_specs=pl.BlockSpec((1, C), lambda i: (0, 0)),
        out_shape=jax.ShapeDtypeStruct((1, C), jnp.float32),
    )(h, w, b.reshape(1, C))


def _layer(xp, w1, b1, w2, b2):
    xf, b2col = _fc1(xp, w1, b1)
    col = jnp.arange(NP, dtype=jnp.int32)
    b2row = jnp.where(col[None, :] < N, b2col.reshape(1, NP), BIGF)
    nbr8 = _knn(xf, b2col, b2row)
    agg, deg = _mp(nbr8, xf)
    kmc = _kmeans(xf)
    return _fc2(agg, deg, kmc, w2, b2)


def kernel(x_0, fc1_w0, fc1_b0, fc2_w0, fc2_b0, fc1_w1, fc1_b1,
           fc2_w1, fc2_b1, lin_w, lin_b):
    xp = jnp.pad(x_0, ((0, NP - N), (0, 0)))
    h = _layer(xp, fc1_w0, fc1_b0, fc2_w0, fc2_b0)
    h = _layer(h, fc1_w1, fc1_b1, fc2_w1, fc2_b1)
    return _pool(h, lin_w, lin_b).reshape(C)
